# baseline (speedup 1.0000x reference)
import functools

import jax
import jax.numpy as jnp
from jax import lax
from jax.experimental import pallas as pl
from jax.experimental.pallas import tpu as pltpu
from jax.experimental.pallas import tpu_sc as plsc

VOCAB = 1000000
EMBED_DIM = 64
BATCH = 16384
N_FIELDS = 26

_INFO = plsc.get_sparse_core_info()
NC, NS = _INFO.num_cores, _INFO.num_subcores
NW = NC * NS
TOTAL = BATCH * N_FIELDS
PER_W = TOTAL // NW
CHUNK = 128
NCHUNK = PER_W // CHUNK

NBLK = VOCAB // 128
TAIL_V = NBLK * 128
TAIL_W = VOCAB - TAIL_V
BASE_BLKS = NBLK // NW
EXTRA = NBLK - BASE_BLKS * NW
PITCH = EMBED_DIM + 1


@functools.partial(
    pl.kernel,
    mesh=plsc.VectorSubcoreMesh(core_axis_name="c", subcore_axis_name="s"),
    out_type=jax.ShapeDtypeStruct((VOCAB * EMBED_DIM,), jnp.float32),
    scratch_types=[
        [pltpu.VMEM((EMBED_DIM, 128), jnp.float32) for _ in range(6)],
        [pltpu.VMEM((128 * EMBED_DIM,), jnp.float32) for _ in range(6)],
        pltpu.SemaphoreType.DMA,
        pltpu.SemaphoreType.DMA,
    ],
    compiler_params=pltpu.CompilerParams(
        use_tc_tiling_on_sc=True, needs_layout_passes=False
    ),
)
def _untile_kernel(tt_hbm, tail_hbm, scratch_hbm, slabs, rows, in_sem, out_sem):
    wid = lax.axis_index("s") * NC + lax.axis_index("c")
    start = wid * BASE_BLKS + lax.min(wid, EXTRA)

    def slab_in(blk, buf):
        col0 = pl.multiple_of(blk * 128, 128)
        pltpu.make_async_copy(
            tt_hbm.at[:, pl.ds(col0, 128)], slabs[buf], in_sem
        ).start()

    def slab_in_wait(buf):
        pltpu.make_async_copy(
            tt_hbm.at[:, pl.ds(0, 128)], slabs[buf], in_sem
        ).wait()

    def rows_out(blk, buf):
        pltpu.make_async_copy(
            rows[buf], scratch_hbm.at[pl.ds(blk * 128 * 64, 128 * 64)], out_sem
        ).start()

    def rows_out_wait():
        pltpu.make_async_copy(
            rows[0], scratch_hbm.at[pl.ds(0, 128 * 64)], out_sem
        ).wait()

    lanes = lax.iota(jnp.int32, 16)

    def transpose_slab(buf):
        @plsc.parallel_loop(0, 64, step=1, unroll=4)
        def _(d0):
            drow = lax.rem(lanes + d0, 64)
            for j in range(8):
                cols = lanes + j * 16
                vec = plsc.load_gather(slabs[buf], [drow, cols])
                plsc.store_scatter(rows[buf], [cols * 64 + drow], vec)

    NMAIN = (BASE_BLKS // 6) * 6
    for k in range(5):
        slab_in(start + k, k)

    def body(i6, _):
        for b in range(6):
            i = i6 * 6 + b
            blk = start + i

            @pl.when(i + 5 < BASE_BLKS)
            def _():
                slab_in(blk + 5, (b + 5) % 6)

            slab_in_wait(b)

            @pl.when(i >= 6)
            def _():
                rows_out_wait()

            transpose_slab(b)
            rows_out(blk, b)
        return 0

    lax.fori_loop(0, NMAIN // 6, body, 0)
    for r in range(BASE_BLKS - NMAIN):
        i = NMAIN + r
        b = i % 6
        slab_in_wait(b)
        rows_out_wait()
        transpose_slab(b)
        rows_out(start + i, b)
    for _k in range(6):
        rows_out_wait()

    @pl.when(wid < EXTRA)
    def _():
        blk = start + BASE_BLKS
        slab_in(blk, 0)
        slab_in_wait(0)
        transpose_slab(0)
        rows_out(blk, 0)
        rows_out_wait()

    @pl.when(wid == NW - 1)
    def _():
        pltpu.make_async_copy(
            tail_hbm, rows[0].at[pl.ds(0, TAIL_W * 64)], in_sem
        ).start()
        pltpu.make_async_copy(
            tail_hbm, rows[0].at[pl.ds(0, TAIL_W * 64)], in_sem
        ).wait()
        pltpu.make_async_copy(
            rows[0].at[pl.ds(0, TAIL_W * 64)],
            scratch_hbm.at[pl.ds(TAIL_V * 64, TAIL_W * 64)],
            out_sem,
        ).start()
        pltpu.make_async_copy(
            rows[0].at[pl.ds(0, TAIL_W * 64)],
            scratch_hbm.at[pl.ds(0, TAIL_W * 64)],
            out_sem,
        ).wait()


@functools.partial(
    pl.kernel,
    mesh=plsc.VectorSubcoreMesh(core_axis_name="c", subcore_axis_name="s"),
    out_type=jax.ShapeDtypeStruct((N_FIELDS, 8, 128, 8, 128), jnp.float32),
    scratch_types=[
        pltpu.VMEM((N_FIELDS, 512), jnp.int32),
        [pltpu.VMEM((CHUNK, EMBED_DIM), jnp.float32) for _ in range(4)],
        [pltpu.VMEM((8, 8, 128), jnp.float32) for _ in range(4)],
        pltpu.SemaphoreType.DMA,
        pltpu.SemaphoreType.DMA,
        pltpu.SemaphoreType.DMA,
    ],
    compiler_params=pltpu.CompilerParams(
        use_tc_tiling_on_sc=False, needs_layout_passes=False
    ),
)
def _gather_kernel(xt_hbm, table_hbm, out5, idx_v, rows, stage, gsem, osem, isem):
    wid = lax.axis_index("s") * NC + lax.axis_index("c")
    b0 = wid * 512
    pltpu.make_async_copy(xt_hbm.at[:, pl.ds(b0, 512)], idx_v, isem).start()
    pltpu.make_async_copy(xt_hbm.at[:, pl.ds(b0, 512)], idx_v, isem).wait()

    NUNIT = N_FIELDS * 4

    def gather(u, buf):
        f, bb = lax.div(u, 4), lax.rem(u, 4)
        pltpu.make_async_copy(
            table_hbm.at[idx_v.at[f, pl.ds(bb * 128, 128)]], rows[buf], gsem
        ).start()

    def gather_wait(buf):
        pltpu.make_async_copy(
            table_hbm.at[idx_v.at[0, pl.ds(0, 128)]], rows[buf], gsem
        ).wait()

    lanes = lax.iota(jnp.int32, 16)

    def transpose(buf):
        @plsc.parallel_loop(0, 64, step=1, unroll=4)
        def _(d0):
            dvec = lax.rem(lanes + d0, 64)
            for j in range(8):
                bvec = lanes + j * 16
                vec = plsc.load_gather(rows[buf], [bvec, dvec])
                plsc.store_scatter(
                    stage[buf],
                    [
                        lax.shift_right_logical(dvec, 3),
                        lax.bitwise_and(dvec, 7),
                        bvec,
                    ],
                    vec,
                )

    def put(u, buf):
        f, bb = lax.div(u, 4), lax.rem(u, 4)
        pltpu.make_async_copy(
            stage[buf], out5.at[f, :, wid * 4 + bb], osem
        ).start()

    def put_wait(buf):
        pltpu.make_async_copy(stage[buf], out5.at[0, :, 0], osem).wait()

    for k in range(3):
        gather(k, k)

    def body(u4, _):
        for b in range(4):
            u = u4 * 4 + b

            @pl.when(u + 3 < NUNIT)
            def _():
                gather(u + 3, (b + 3) % 4)

            gather_wait(b)

            @pl.when(u >= 4)
            def _():
                put_wait(b)

            transpose(b)
            put(u, b)
        return 0

    lax.fori_loop(0, NUNIT // 4, body, 0)
    for _k in range(4):
        put_wait(0)


def kernel(x, table):
    tt = table.T
    tail = table[TAIL_V:].reshape(-1)
    scratch = _untile_kernel(tt, tail)
    table_rm = scratch.reshape(VOCAB, EMBED_DIM)
    out5 = _gather_kernel(x.T, table_rm)
    return out5.transpose(2, 4, 0, 1, 3).reshape(BATCH, N_FIELDS, EMBED_DIM)


# speedup vs baseline: 1.0016x; 1.0016x over previous
import functools

import jax
import jax.numpy as jnp
from jax import lax
from jax.experimental import pallas as pl
from jax.experimental.pallas import tpu as pltpu
from jax.experimental.pallas import tpu_sc as plsc

VOCAB = 1000000
EMBED_DIM = 64
BATCH = 16384
N_FIELDS = 26

_INFO = plsc.get_sparse_core_info()
NC, NS = _INFO.num_cores, _INFO.num_subcores
NW = NC * NS
TOTAL = BATCH * N_FIELDS
CHUNK = 128

NBLK = VOCAB // 128
TAIL_V = NBLK * 128
TAIL_W = VOCAB - TAIL_V
BASE_BLKS = NBLK // NW
EXTRA = NBLK - BASE_BLKS * NW


@functools.partial(
    pl.kernel,
    mesh=plsc.VectorSubcoreMesh(core_axis_name="c", subcore_axis_name="s"),
    out_type=jax.ShapeDtypeStruct((VOCAB * EMBED_DIM,), jnp.float32),
    scratch_types=[
        [pltpu.VMEM((EMBED_DIM, 128), jnp.float32) for _ in range(4)],
        [pltpu.VMEM((128 * EMBED_DIM,), jnp.float32) for _ in range(4)],
        pltpu.SemaphoreType.DMA,
        pltpu.SemaphoreType.DMA,
    ],
    compiler_params=pltpu.CompilerParams(
        use_tc_tiling_on_sc=True, needs_layout_passes=False
    ),
)
def _untile_kernel(tt_hbm, tail_hbm, scratch_hbm, slabs, rows, in_sem, out_sem):
    wid = lax.axis_index("s") * NC + lax.axis_index("c")
    start = wid * BASE_BLKS + lax.min(wid, EXTRA)

    def slab_in(blk, buf):
        col0 = pl.multiple_of(blk * 128, 128)
        pltpu.make_async_copy(
            tt_hbm.at[:, pl.ds(col0, 128)], slabs[buf], in_sem
        ).start()

    def slab_in_wait(buf):
        pltpu.make_async_copy(
            tt_hbm.at[:, pl.ds(0, 128)], slabs[buf], in_sem
        ).wait()

    def rows_out(blk, buf):
        pltpu.make_async_copy(
            rows[buf], scratch_hbm.at[pl.ds(blk * 128 * 64, 128 * 64)], out_sem
        ).start()

    def rows_out_wait():
        pltpu.make_async_copy(
            rows[0], scratch_hbm.at[pl.ds(0, 128 * 64)], out_sem
        ).wait()

    lanes = lax.iota(jnp.int32, 16)

    def transpose_slab(buf):
        @plsc.parallel_loop(0, 64, step=1, unroll=4)
        def _(d0):
            drow = lax.rem(lanes + d0, 64)
            for j in range(8):
                cols = lanes + j * 16
                vec = plsc.load_gather(slabs[buf], [drow, cols])
                plsc.store_scatter(rows[buf], [cols * 64 + drow], vec)

    for k in range(3):
        slab_in(start + k, k)

    def body(i4, _):
        for b in range(4):
            i = i4 * 4 + b
            blk = start + i

            @pl.when(i + 3 < BASE_BLKS)
            def _():
                slab_in(blk + 3, (b + 3) % 4)

            slab_in_wait(b)

            @pl.when(i >= 4)
            def _():
                rows_out_wait()

            transpose_slab(b)
            rows_out(blk, b)
        return 0

    lax.fori_loop(0, BASE_BLKS // 4, body, 0)
    for _k in range(4):
        rows_out_wait()

    @pl.when(wid < EXTRA)
    def _():
        blk = start + BASE_BLKS
        slab_in(blk, 0)
        slab_in_wait(0)
        transpose_slab(0)
        rows_out(blk, 0)
        rows_out_wait()

    @pl.when(wid == NW - 1)
    def _():
        pltpu.make_async_copy(
            tail_hbm, rows[0].at[pl.ds(0, TAIL_W * 64)], in_sem
        ).start()
        pltpu.make_async_copy(
            tail_hbm, rows[0].at[pl.ds(0, TAIL_W * 64)], in_sem
        ).wait()
        pltpu.make_async_copy(
            rows[0].at[pl.ds(0, TAIL_W * 64)],
            scratch_hbm.at[pl.ds(TAIL_V * 64, TAIL_W * 64)],
            out_sem,
        ).start()
        pltpu.make_async_copy(
            rows[0].at[pl.ds(0, TAIL_W * 64)],
            scratch_hbm.at[pl.ds(0, TAIL_W * 64)],
            out_sem,
        ).wait()


@functools.partial(
    pl.kernel,
    mesh=plsc.VectorSubcoreMesh(core_axis_name="c", subcore_axis_name="s"),
    out_type=jax.ShapeDtypeStruct((N_FIELDS, 8, 128, 8, 128), jnp.float32),
    scratch_types=[
        pltpu.VMEM((N_FIELDS, 512), jnp.int32),
        [pltpu.VMEM((CHUNK, EMBED_DIM), jnp.float32) for _ in range(4)],
        [pltpu.VMEM((8, 8, 128), jnp.float32) for _ in range(4)],
        pltpu.SemaphoreType.DMA,
        pltpu.SemaphoreType.DMA,
        pltpu.SemaphoreType.DMA,
    ],
    compiler_params=pltpu.CompilerParams(
        use_tc_tiling_on_sc=False, needs_layout_passes=False
    ),
)
def _gather_kernel(xt_hbm, table_hbm, out5, idx_v, rows, stage, gsem, osem, isem):
    wid = lax.axis_index("s") * NC + lax.axis_index("c")
    b0 = wid * 512
    pltpu.make_async_copy(xt_hbm.at[:, pl.ds(b0, 512)], idx_v, isem).start()
    pltpu.make_async_copy(xt_hbm.at[:, pl.ds(b0, 512)], idx_v, isem).wait()

    NUNIT = N_FIELDS * 4

    def gather(u, buf):
        f, bb = lax.div(u, 4), lax.rem(u, 4)
        pltpu.make_async_copy(
            table_hbm.at[idx_v.at[f, pl.ds(bb * 128, 128)]], rows[buf], gsem
        ).start()

    def gather_wait(buf):
        pltpu.make_async_copy(
            table_hbm.at[idx_v.at[0, pl.ds(0, 128)]], rows[buf], gsem
        ).wait()

    lanes = lax.iota(jnp.int32, 16)

    def transpose(buf):
        @plsc.parallel_loop(0, 64, step=1, unroll=4)
        def _(d0):
            dvec = lax.rem(lanes + d0, 64)
            for j in range(8):
                bvec = lanes + j * 16
                vec = plsc.load_gather(rows[buf], [bvec, dvec])
                plsc.store_scatter(
                    stage[buf],
                    [
                        lax.shift_right_logical(dvec, 3),
                        lax.bitwise_and(dvec, 7),
                        bvec,
                    ],
                    vec,
                )

    def put(u, buf):
        f, bb = lax.div(u, 4), lax.rem(u, 4)
        pltpu.make_async_copy(
            stage[buf], out5.at[f, :, wid * 4 + bb], osem
        ).start()

    def put_wait(buf):
        pltpu.make_async_copy(stage[buf], out5.at[0, :, 0], osem).wait()

    for k in range(3):
        gather(k, k)

    def body(u4, _):
        for b in range(4):
            u = u4 * 4 + b

            @pl.when(u + 3 < NUNIT)
            def _():
                gather(u + 3, (b + 3) % 4)

            gather_wait(b)

            @pl.when(u >= 4)
            def _():
                put_wait(b)

            transpose(b)
            put(u, b)
        return 0

    lax.fori_loop(0, NUNIT // 4, body, 0)
    for _k in range(4):
        put_wait(0)


def kernel(x, table):
    tt = table.T
    tail = table[TAIL_V:].reshape(-1)
    scratch = _untile_kernel(tt, tail)
    table_rm = scratch.reshape(VOCAB, EMBED_DIM)
    out5 = _gather_kernel(x.T, table_rm)
    return out5.transpose(2, 4, 0, 1, 3).reshape(BATCH, N_FIELDS, EMBED_DIM)
